# baseline (speedup 1.0000x reference)
import functools

import jax
import jax.numpy as jnp
from jax import lax
from jax.experimental import pallas as pl
from jax.experimental.pallas import tpu as pltpu
from jax.experimental.pallas import tpu_sc as plsc

_ANCHORS = (
    (1.3221, 1.73145),
    (3.19275, 4.00944),
    (5.05587, 8.09892),
    (9.47112, 4.84053),
    (11.2364, 10.0071),
)
_G = 26
_NPIX = _G * _G
_HPIX = _NPIX // 2
_NA = 5
_NCH = 85
_STRIDE = 32.0
_NB = 32
_HVEC = 22
_LAST_P0 = _HPIX - 16
_RPAD = 432
_PPAD = 768
_CROWS = 96
_WROWS = 344
_ROWS_PAD = 3384
_COLS_PAD = 128
_CSTARTS = (0, 16, 32, 48, 64, _NCH - 16)

_mesh = plsc.VectorSubcoreMesh(core_axis_name="c", subcore_axis_name="s")


@functools.partial(
    pl.kernel,
    mesh=_mesh,
    out_type=jax.ShapeDtypeStruct((_NB, _ROWS_PAD, _COLS_PAD), jnp.float32),
    scratch_types=[
        pltpu.VMEM((_CROWS, _PPAD), jnp.float32),
        pltpu.VMEM((_WROWS, _COLS_PAD), jnp.float32),
        pltpu.VMEM((8, _COLS_PAD), jnp.float32),
    ],
    compiler_params=pltpu.CompilerParams(
        use_tc_tiling_on_sc=False, needs_layout_passes=False
    ),
)
def _sc_decode(z_hbm, out_hbm, in_v, out_v, stash_v):
    wid = lax.axis_index("s") * 2 + lax.axis_index("c")
    iota = lax.iota(jnp.int32, 16)

    for a in range(_NA):
        row0 = (_NCH * a) // 8 * 8
        cph = _NCH * a - row0
        pltpu.sync_copy(
            z_hbm.at[wid, pl.ds(row0, _CROWS), pl.ds(0, _PPAD)], in_v
        )

        aw32 = jnp.float32(_ANCHORS[a][0] * _STRIDE)
        ah32 = jnp.float32(_ANCHORS[a][1] * _STRIDE)

        for h in range(2):
            o = _NPIX * a + _HPIX * h
            ph = o % 8
            base = _HPIX * h

            def pix_block(
                j, carry, cph=cph, ph=ph, base=base, aw32=aw32, ah32=ah32
            ):
                p0 = base + jnp.minimum(j * 16, _LAST_P0)
                pv = p0 + iota
                rv = pv - base + ph
                ii = pv // _G
                jj = pv % _G
                gx32 = jj.astype(jnp.float32) * _STRIDE
                gy32 = ii.astype(jnp.float32) * _STRIDE

                def splat(c):
                    return jnp.full((16,), c, jnp.int32)

                def sig(c):
                    v = in_v[cph + c, pl.ds(p0, 16)]
                    return 1.0 / (1.0 + jnp.exp(-v))

                def expo(c):
                    v = in_v[cph + c, pl.ds(p0, 16)]
                    return jnp.exp(v)

                plsc.store_scatter(
                    out_v, [rv, splat(0)], sig(0) * _STRIDE + gx32
                )
                plsc.store_scatter(
                    out_v, [rv, splat(1)], sig(1) * _STRIDE + gy32
                )
                plsc.store_scatter(out_v, [rv, splat(2)], expo(2) * aw32)
                plsc.store_scatter(out_v, [rv, splat(3)], expo(3) * ah32)

                @plsc.parallel_loop(4, _NCH, 1, unroll=4)
                def sig_rows(c):
                    plsc.store_scatter(out_v, [rv, splat(c)], sig(c))

                return carry

            z = lax.fori_loop(0, _HVEC, pix_block, 0)
            del z

            if ph:
                for r in range(ph):
                    for c0 in _CSTARTS:
                        out_v[r, pl.ds(c0, 16)] = stash_v[r, pl.ds(c0, 16)]

            nxt = (o + _HPIX) % 8
            if not (a == _NA - 1 and h == 1) and nxt:
                for r in range(nxt):
                    src = ph + _HPIX - nxt + r
                    for c0 in _CSTARTS:
                        stash_v[r, pl.ds(c0, 16)] = out_v[src, pl.ds(c0, 16)]

            pltpu.sync_copy(
                out_v,
                out_hbm.at[wid, pl.ds(o - ph, _WROWS), pl.ds(0, _COLS_PAD)],
            )


def kernel(x):
    B = x.shape[0]
    z = jnp.pad(
        x.reshape(B, _NA * _NCH, _NPIX),
        ((0, 0), (0, _RPAD - _NA * _NCH), (0, _PPAD - _NPIX)),
    )
    out = _sc_decode(z)
    return out[:, : _NA * _NPIX, :_NCH]


# speedup vs baseline: 1.0874x; 1.0874x over previous
import functools

import jax
import jax.numpy as jnp
from jax import lax
from jax.experimental import pallas as pl
from jax.experimental.pallas import tpu as pltpu
from jax.experimental.pallas import tpu_sc as plsc

_ANCHORS = (
    (1.3221, 1.73145),
    (3.19275, 4.00944),
    (5.05587, 8.09892),
    (9.47112, 4.84053),
    (11.2364, 10.0071),
)
_G = 26
_NPIX = _G * _G
_HPIX = _NPIX // 2
_NA = 5
_NCH = 85
_STRIDE = 32.0
_NB = 32
_HVEC = 22
_LAST_P0 = _HPIX - 16
_RPAD = 432
_PPAD = 768
_CROWS = 96
_WROWS = 344
_ROWS_PAD = 3384
_COLS_PAD = 128
_CSTARTS = (0, 16, 32, 48, 64, _NCH - 16)

_mesh = plsc.VectorSubcoreMesh(core_axis_name="c", subcore_axis_name="s")


@functools.partial(
    pl.kernel,
    mesh=_mesh,
    out_type=jax.ShapeDtypeStruct((_NB, _ROWS_PAD, _COLS_PAD), jnp.float32),
    scratch_types=[
        pltpu.VMEM((_CROWS, _PPAD), jnp.float32),
        pltpu.VMEM((_WROWS, _COLS_PAD), jnp.float32),
        pltpu.VMEM((8, _COLS_PAD), jnp.float32),
    ],
    compiler_params=pltpu.CompilerParams(
        use_tc_tiling_on_sc=True, needs_layout_passes=False
    ),
)
def _sc_decode(z_hbm, out_hbm, in_v, out_v, stash_v):
    wid = lax.axis_index("s") * 2 + lax.axis_index("c")
    iota = lax.iota(jnp.int32, 16)

    for a in range(_NA):
        row0 = (_NCH * a) // 8 * 8
        cph = _NCH * a - row0
        pltpu.sync_copy(
            z_hbm.at[wid, pl.ds(row0, _CROWS), pl.ds(0, _PPAD)], in_v
        )

        aw32 = jnp.float32(_ANCHORS[a][0] * _STRIDE)
        ah32 = jnp.float32(_ANCHORS[a][1] * _STRIDE)

        for h in range(2):
            o = _NPIX * a + _HPIX * h
            ph = o % 8
            base = _HPIX * h

            def pix_block(
                j, carry, cph=cph, ph=ph, base=base, aw32=aw32, ah32=ah32
            ):
                p0 = base + jnp.minimum(j * 16, _LAST_P0)
                pv = p0 + iota
                rv = pv - base + ph
                ii = pv // _G
                jj = pv % _G
                gx32 = jj.astype(jnp.float32) * _STRIDE
                gy32 = ii.astype(jnp.float32) * _STRIDE

                def splat(c):
                    return jnp.full((16,), c, jnp.int32)

                def sig(c):
                    v = in_v[cph + c, pl.ds(p0, 16)]
                    return 1.0 / (1.0 + jnp.exp(-v))

                def expo(c):
                    v = in_v[cph + c, pl.ds(p0, 16)]
                    return jnp.exp(v)

                plsc.store_scatter(
                    out_v, [rv, splat(0)], sig(0) * _STRIDE + gx32
                )
                plsc.store_scatter(
                    out_v, [rv, splat(1)], sig(1) * _STRIDE + gy32
                )
                plsc.store_scatter(out_v, [rv, splat(2)], expo(2) * aw32)
                plsc.store_scatter(out_v, [rv, splat(3)], expo(3) * ah32)

                @plsc.parallel_loop(4, _NCH, 1, unroll=4)
                def sig_rows(c):
                    plsc.store_scatter(out_v, [rv, splat(c)], sig(c))

                return carry

            z = lax.fori_loop(0, _HVEC, pix_block, 0)
            del z

            if ph:
                for r in range(ph):
                    for c0 in _CSTARTS:
                        out_v[r, pl.ds(c0, 16)] = stash_v[r, pl.ds(c0, 16)]

            nxt = (o + _HPIX) % 8
            if not (a == _NA - 1 and h == 1) and nxt:
                for r in range(nxt):
                    src = ph + _HPIX - nxt + r
                    for c0 in _CSTARTS:
                        stash_v[r, pl.ds(c0, 16)] = out_v[src, pl.ds(c0, 16)]

            pltpu.sync_copy(
                out_v,
                out_hbm.at[wid, pl.ds(o - ph, _WROWS), pl.ds(0, _COLS_PAD)],
            )


def kernel(x):
    B = x.shape[0]
    z = jnp.pad(
        x.reshape(B, _NA * _NCH, _NPIX),
        ((0, 0), (0, _RPAD - _NA * _NCH), (0, _PPAD - _NPIX)),
    )
    out = _sc_decode(z)
    return out[:, : _NA * _NPIX, :_NCH]
